# baseline (speedup 1.0000x reference)
import functools

import jax
import jax.numpy as jnp
from jax import lax
from jax.experimental import pallas as pl
from jax.experimental.pallas import tpu as pltpu
from jax.experimental.pallas import tpu_sc as plsc

BATCH = 16384
HIDDEN = 64
PADW = 128
ENT_TOTAL = 1000000
REL_TOTAL = 1000
NC = 2
NS = 16
NW = NC * NS
PER_W = BATCH // NW
CHUNK = 16
NCHUNK = PER_W // CHUNK
LANES = 16
MARGIN = 1.0

K1_BLK = 320
K1_NBLK = ENT_TOTAL // K1_BLK
K1_KMAX = -(-K1_NBLK // NW)
REL_BLK = 200
F32 = jnp.float32
I32 = jnp.int32


def _mesh():
    return plsc.VectorSubcoreMesh(
        core_axis_name="c", subcore_axis_name="s",
        num_cores=NC, num_subcores=NS)


def _params():
    return pltpu.CompilerParams(needs_layout_passes=False,
                                use_tc_tiling_on_sc=True)


def _wid():
    return lax.axis_index("s") * NC + lax.axis_index("c")


def _shuffle(src64, dst128, nrows):
    def row(r, _):
        a = src64.at[2 * r]
        c = src64.at[2 * r + 1]
        d = dst128.at[r]
        for k in range(0, HIDDEN, LANES):
            d[pl.ds(k, LANES)] = a[pl.ds(k, LANES)]
            d[pl.ds(HIDDEN + k, LANES)] = c[pl.ds(k, LANES)]
        return 0

    lax.fori_loop(0, nrows // 2, row, 0, unroll=2)


def _stage_body(ent_hbm, rel_hbm, nv_hbm, s_ent, s_rel, s_nv,
                b64a, b64b, b128a, b128b,
                sem_ia, sem_ib, sem_oa, sem_ob):
    wid = _wid()

    def rel_block(tbl, dst, w0):
        @pl.when((wid >= w0) & (wid < w0 + 3))
        def _():
            w = wid - w0
            pltpu.sync_copy(tbl.at[pl.ds(w * 320, 320)], b64a)
            _shuffle(b64a, b128a, 320)
            pltpu.sync_copy(b128a, dst.at[pl.ds(w * 160, 160)])

        @pl.when(wid == w0 + 3)
        def _():
            pltpu.sync_copy(tbl.at[pl.ds(960, 40)], b64a.at[0:40])
            _shuffle(b64a.at[0:48], b128a.at[0:24], 48)
            pltpu.sync_copy(b128a.at[0:24], dst.at[pl.ds(480, 24)])

    rel_block(rel_hbm, s_rel, 0)
    rel_block(nv_hbm, s_nv, 4)

    def blk_of(k):
        return k * NW + wid

    def in_copy(blk, b64, sem):
        return pltpu.make_async_copy(
            ent_hbm.at[pl.ds(blk * K1_BLK, K1_BLK)], b64, sem)

    def out_copy(blk, b128, sem):
        return pltpu.make_async_copy(
            b128, s_ent.at[pl.ds(blk * (K1_BLK // 2), K1_BLK // 2)], sem)

    def start_in(k, b64, sem):
        @pl.when(blk_of(k) < K1_NBLK)
        def _():
            in_copy(blk_of(k), b64, sem).start()

    def slot(k, b64, b128, sem_i, sem_o):
        blk = blk_of(k)

        @pl.when(blk < K1_NBLK)
        def _():
            in_copy(blk, b64, sem_i).wait()
            @pl.when(k >= 2)
            def _():
                out_copy(blk - 2 * NW, b128, sem_o).wait()
            _shuffle(b64, b128, K1_BLK)
            out_copy(blk, b128, sem_o).start()

    start_in(0, b64a, sem_ia)
    start_in(1, b64b, sem_ib)

    def pair(kk, _):
        ka = 2 * kk
        slot(ka, b64a, b128a, sem_ia, sem_oa)
        start_in(ka + 2, b64a, sem_ia)
        slot(ka + 1, b64b, b128b, sem_ib, sem_ob)
        start_in(ka + 3, b64b, sem_ib)
        return 0

    lax.fori_loop(0, K1_KMAX // 2, pair, 0)

    def drain(k, b128, sem_o):
        @pl.when(blk_of(k) < K1_NBLK)
        def _():
            out_copy(blk_of(k), b128, sem_o).wait()

    drain(K1_KMAX - 2, b128a, sem_oa)
    drain(K1_KMAX - 1, b128b, sem_ob)


def _rsqrt16(x):
    x = jnp.maximum(x, F32(1e-12))
    i = plsc.bitcast(x, I32)
    y = plsc.bitcast(I32(0x5F3759DF) - (i >> 1), F32)
    for _ in range(3):
        y = y * (F32(1.5) - F32(0.5) * x * y * y)
    return y


def _main_body(s_ent, s_rel, s_nv,
               ph_hbm, pt_hbm, pr_hbm, nh_hbm, nt_hbm, nr_hbm,
               out_hbm,
               i_ph, i_pt, i_pr, i_nh, i_nt, i_nr,
               k_ph, k_pt, k_pr, k_nh, k_nt, k_nr,
               b_ph, b_pt, b_nh, b_nt,
               b_pr, b_pn, b_nr, b_nn,
               out_stage, sem_a, sem_b):
    wid = _wid()
    base = wid * PER_W

    for src, dst in ((ph_hbm, i_ph), (pt_hbm, i_pt), (pr_hbm, i_pr),
                     (nh_hbm, i_nh), (nt_hbm, i_nt), (nr_hbm, i_nr)):
        pltpu.sync_copy(src.at[pl.ds(base, PER_W)], dst)

    def shift(k, _):
        sl = pl.ds(k * CHUNK, CHUNK)
        for i_r, k_r in ((i_ph, k_ph), (i_pt, k_pt), (i_pr, k_pr),
                         (i_nh, k_nh), (i_nt, k_nt), (i_nr, k_nr)):
            k_r[sl] = i_r[sl] >> 1
        return 0

    lax.fori_loop(0, NCHUNK, shift, 0, unroll=4)

    sems = (sem_a, sem_b)
    lanes = lax.iota(I32, LANES)
    bufs = (b_ph, b_pt, b_nh, b_nt, b_pr, b_pn, b_nr, b_nn)

    def copies(g, b):
        sem = sems[b]
        sl = pl.ds(g * CHUNK, CHUNK)
        srcs = (s_ent.at[k_ph.at[sl]], s_ent.at[k_pt.at[sl]],
                s_ent.at[k_nh.at[sl]], s_ent.at[k_nt.at[sl]],
                s_rel.at[k_pr.at[sl]], s_nv.at[k_pr.at[sl]],
                s_rel.at[k_nr.at[sl]], s_nv.at[k_nr.at[sl]])
        return [pltpu.make_async_copy(src, dst.at[b], sem)
                for src, dst in zip(srcs, bufs)]

    def issue(g, b):
        for cp in copies(g, b):
            cp.start()

    def compute(g, b):
        sl = pl.ds(g * CHUNK, CHUNK)
        c_ph = (i_ph[sl] & 1) << 6
        c_pt = (i_pt[sl] & 1) << 6
        c_nh = (i_nh[sl] & 1) << 6
        c_nt = (i_nt[sl] & 1) << 6
        c_pr = (i_pr[sl] & 1) << 6
        c_nr = (i_nr[sl] & 1) << 6
        rp_h, rp_t, rn_h, rn_t = b_ph.at[b], b_pt.at[b], b_nh.at[b], b_nt.at[b]
        rp_r, rp_n, rn_r, rn_n = b_pr.at[b], b_pn.at[b], b_nr.at[b], b_nn.at[b]

        def pass1(j, acc):
            col = jnp.full((LANES,), j, I32)
            (phh, ptt, prr, pnn, phn, ptn,
             qhh, qtt, qrr, qnn, qhn, qtn) = acc
            ph = plsc.load_gather(rp_h, [lanes, c_ph + col])
            pt = plsc.load_gather(rp_t, [lanes, c_pt + col])
            pr = plsc.load_gather(rp_r, [lanes, c_pr + col])
            pn = plsc.load_gather(rp_n, [lanes, c_pr + col])
            nh = plsc.load_gather(rn_h, [lanes, c_nh + col])
            nt = plsc.load_gather(rn_t, [lanes, c_nt + col])
            nr = plsc.load_gather(rn_r, [lanes, c_nr + col])
            nn = plsc.load_gather(rn_n, [lanes, c_nr + col])
            return (phh + ph * ph, ptt + pt * pt, prr + pr * pr,
                    pnn + pn * pn, phn + ph * pn, ptn + pt * pn,
                    qhh + nh * nh, qtt + nt * nt, qrr + nr * nr,
                    qnn + nn * nn, qhn + nh * nn, qtn + nt * nn)

        z = jnp.zeros((LANES,), F32)
        (phh, ptt, prr, pnn, phn, ptn,
         qhh, qtt, qrr, qnn, qhn, qtn) = lax.fori_loop(
             0, HIDDEN, pass1, (z,) * 12, unroll=8)

        p_ih, p_it, p_ir = _rsqrt16(phh), _rsqrt16(ptt), _rsqrt16(prr)
        p_in = _rsqrt16(pnn)
        q_ih, q_it, q_ir = _rsqrt16(qhh), _rsqrt16(qtt), _rsqrt16(qrr)
        q_in = _rsqrt16(qnn)
        p_c = (phn * p_ih - ptn * p_it) * p_in * p_in
        q_c = (qhn * q_ih - qtn * q_it) * q_in * q_in

        def pass2(j, acc):
            col = jnp.full((LANES,), j, I32)
            accp, accn = acc
            ph = plsc.load_gather(rp_h, [lanes, c_ph + col])
            pt = plsc.load_gather(rp_t, [lanes, c_pt + col])
            pr = plsc.load_gather(rp_r, [lanes, c_pr + col])
            pn = plsc.load_gather(rp_n, [lanes, c_pr + col])
            nh = plsc.load_gather(rn_h, [lanes, c_nh + col])
            nt = plsc.load_gather(rn_t, [lanes, c_nt + col])
            nr = plsc.load_gather(rn_r, [lanes, c_nr + col])
            nn = plsc.load_gather(rn_n, [lanes, c_nr + col])
            vp = ph * p_ih + pr * p_ir - pt * p_it - p_c * pn
            vn = nh * q_ih + nr * q_ir - nt * q_it - q_c * nn
            return (accp + jnp.abs(vp), accn + jnp.abs(vn))

        accp, accn = lax.fori_loop(0, HIDDEN, pass2, (z, z), unroll=8)
        return jnp.maximum(accp - accn + F32(MARGIN), F32(0.0))

    issue(0, 0)
    issue(1, 1)

    def pair(g2, loss):
        ga = g2 * 2
        for cp in copies(ga, 0):
            cp.wait()
        loss = loss + compute(ga, 0)

        @pl.when(ga + 2 < NCHUNK)
        def _():
            issue(ga + 2, 0)

        for cp in copies(ga + 1, 1):
            cp.wait()
        loss = loss + compute(ga + 1, 1)

        @pl.when(ga + 3 < NCHUNK)
        def _():
            issue(ga + 3, 1)
        return loss

    loss_acc = lax.fori_loop(0, NCHUNK // 2, pair, jnp.zeros((LANES,), F32))

    total = jnp.sum(loss_acc)
    out_stage[...] = jnp.where(lanes == 0, total, F32(0.0))
    pltpu.sync_copy(out_stage, out_hbm.at[pl.ds(wid * LANES, LANES)])


@jax.jit
def _launch(ent, rel, nv, ph, pt, pr, nh, nt, nr):
    stage = pl.kernel(
        _stage_body,
        out_type=(jax.ShapeDtypeStruct((ENT_TOTAL // 2, PADW), F32),
                  jax.ShapeDtypeStruct((512, PADW), F32),
                  jax.ShapeDtypeStruct((512, PADW), F32)),
        mesh=_mesh(),
        compiler_params=_params(),
        scratch_types=[pltpu.VMEM((K1_BLK, HIDDEN), F32)] * 2
        + [pltpu.VMEM((K1_BLK // 2, PADW), F32)] * 2
        + [pltpu.SemaphoreType.DMA] * 4,
    )
    s_ent, s_rel, s_nv = stage(ent, rel, nv)

    main = pl.kernel(
        _main_body,
        out_type=jax.ShapeDtypeStruct((NW * LANES,), F32),
        mesh=_mesh(),
        compiler_params=_params(),
        scratch_types=[pltpu.VMEM((PER_W,), I32)] * 12
        + [pltpu.VMEM((2, CHUNK, PADW), F32)] * 8
        + [pltpu.VMEM((LANES,), F32),
           pltpu.SemaphoreType.DMA, pltpu.SemaphoreType.DMA],
    )
    return main(s_ent, s_rel, s_nv, ph, pt, pr, nh, nt, nr)


def kernel(pos_h, pos_t, pos_r, neg_h, neg_t, neg_r,
           ent_embeddings, rel_embeddings, normal_vectors):
    partials = _launch(
        ent_embeddings, rel_embeddings, normal_vectors,
        pos_h.astype(I32), pos_t.astype(I32), pos_r.astype(I32),
        neg_h.astype(I32), neg_t.astype(I32), neg_r.astype(I32))
    return jnp.sum(partials)


# speedup vs baseline: 1.2241x; 1.2241x over previous
import functools

import jax
import jax.numpy as jnp
from jax import lax
from jax.experimental import pallas as pl
from jax.experimental.pallas import tpu as pltpu
from jax.experimental.pallas import tpu_sc as plsc

BATCH = 16384
HIDDEN = 64
PADW = 128
ENT_TOTAL = 1000000
REL_TOTAL = 1000
NC = 2
NS = 16
NW = NC * NS
PER_W = BATCH // NW
CHUNK = 16
NCHUNK = PER_W // CHUNK
LANES = 16
MARGIN = 1.0
F32 = jnp.float32
I32 = jnp.int32


def _rsqrt16(x):
    x = jnp.maximum(x, F32(1e-12))
    i = plsc.bitcast(x, I32)
    y = plsc.bitcast(I32(0x5F3759DF) - (i >> 1), F32)
    for _ in range(3):
        y = y * (F32(1.5) - F32(0.5) * x * y * y)
    return y


def _main_body(s_ent, s_rel, s_nv,
               ph_hbm, pt_hbm, pr_hbm, nh_hbm, nt_hbm, nr_hbm,
               out_hbm,
               i_ph, i_pt, i_pr, i_nh, i_nt, i_nr,
               k_ph, k_pt, k_pr, k_nh, k_nt, k_nr,
               b_ph, b_pt, b_nh, b_nt,
               b_pr, b_pn, b_nr, b_nn,
               out_stage, sem_a, sem_b):
    wid = lax.axis_index("s") * NC + lax.axis_index("c")
    base = wid * PER_W

    for src, dst in ((ph_hbm, i_ph), (pt_hbm, i_pt), (pr_hbm, i_pr),
                     (nh_hbm, i_nh), (nt_hbm, i_nt), (nr_hbm, i_nr)):
        pltpu.sync_copy(src.at[pl.ds(base, PER_W)], dst)

    def shift(k, _):
        sl = pl.ds(k * CHUNK, CHUNK)
        for i_r, k_r in ((i_ph, k_ph), (i_pt, k_pt), (i_pr, k_pr),
                         (i_nh, k_nh), (i_nt, k_nt), (i_nr, k_nr)):
            k_r[sl] = i_r[sl] >> 1
        return 0

    lax.fori_loop(0, NCHUNK, shift, 0, unroll=4)

    sems = (sem_a, sem_b)
    lanes = lax.iota(I32, LANES)
    lane_base = lanes * I32(PADW)
    bufs = (b_ph, b_pt, b_nh, b_nt, b_pr, b_pn, b_nr, b_nn)

    def copies(g, b):
        sem = sems[b]
        sl = pl.ds(g * CHUNK, CHUNK)
        srcs = (s_ent.at[k_ph.at[sl]], s_ent.at[k_pt.at[sl]],
                s_ent.at[k_nh.at[sl]], s_ent.at[k_nt.at[sl]],
                s_rel.at[k_pr.at[sl]], s_nv.at[k_pr.at[sl]],
                s_rel.at[k_nr.at[sl]], s_nv.at[k_nr.at[sl]])
        return [pltpu.make_async_copy(src, dst.at[b], sem)
                for src, dst in zip(srcs, bufs)]

    def issue(g, b):
        for cp in copies(g, b):
            cp.start()

    def compute(g, b):
        sl = pl.ds(g * CHUNK, CHUNK)
        a_ph = (i_ph[sl] & 1) << 6
        a_pt = (i_pt[sl] & 1) << 6
        a_nh = (i_nh[sl] & 1) << 6
        a_nt = (i_nt[sl] & 1) << 6
        a_pr = (i_pr[sl] & 1) << 6
        a_nr = (i_nr[sl] & 1) << 6
        rp_h, rp_t, rn_h, rn_t = b_ph.at[b], b_pt.at[b], b_nh.at[b], b_nt.at[b]
        rp_r, rp_n, rn_r, rn_n = b_pr.at[b], b_pn.at[b], b_nr.at[b], b_nn.at[b]

        def pass1(j, acc):
            (phh, ptt, prr, pnn, phn, ptn,
             qhh, qtt, qrr, qnn, qhn, qtn) = acc
            ph = plsc.load_gather(rp_h, [lanes, a_ph + j])
            pt = plsc.load_gather(rp_t, [lanes, a_pt + j])
            pr = plsc.load_gather(rp_r, [lanes, a_pr + j])
            pn = plsc.load_gather(rp_n, [lanes, a_pr + j])
            nh = plsc.load_gather(rn_h, [lanes, a_nh + j])
            nt = plsc.load_gather(rn_t, [lanes, a_nt + j])
            nr = plsc.load_gather(rn_r, [lanes, a_nr + j])
            nn = plsc.load_gather(rn_n, [lanes, a_nr + j])
            return (phh + ph * ph, ptt + pt * pt, prr + pr * pr,
                    pnn + pn * pn, phn + ph * pn, ptn + pt * pn,
                    qhh + nh * nh, qtt + nt * nt, qrr + nr * nr,
                    qnn + nn * nn, qhn + nh * nn, qtn + nt * nn)

        z = jnp.zeros((LANES,), F32)
        (phh, ptt, prr, pnn, phn, ptn,
         qhh, qtt, qrr, qnn, qhn, qtn) = lax.fori_loop(
             0, HIDDEN, pass1, (z,) * 12, unroll=8)

        p_ih, p_it, p_ir = _rsqrt16(phh), _rsqrt16(ptt), _rsqrt16(prr)
        p_in = _rsqrt16(pnn)
        q_ih, q_it, q_ir = _rsqrt16(qhh), _rsqrt16(qtt), _rsqrt16(qrr)
        q_in = _rsqrt16(qnn)
        p_c = (phn * p_ih - ptn * p_it) * p_in * p_in
        q_c = (qhn * q_ih - qtn * q_it) * q_in * q_in

        def pass2(j, acc):
            accp, accn = acc
            ph = plsc.load_gather(rp_h, [lanes, a_ph + j])
            pt = plsc.load_gather(rp_t, [lanes, a_pt + j])
            pr = plsc.load_gather(rp_r, [lanes, a_pr + j])
            pn = plsc.load_gather(rp_n, [lanes, a_pr + j])
            nh = plsc.load_gather(rn_h, [lanes, a_nh + j])
            nt = plsc.load_gather(rn_t, [lanes, a_nt + j])
            nr = plsc.load_gather(rn_r, [lanes, a_nr + j])
            nn = plsc.load_gather(rn_n, [lanes, a_nr + j])
            vp = ph * p_ih + pr * p_ir - pt * p_it - p_c * pn
            vn = nh * q_ih + nr * q_ir - nt * q_it - q_c * nn
            return (accp + jnp.abs(vp), accn + jnp.abs(vn))

        accp, accn = lax.fori_loop(0, HIDDEN, pass2, (z, z), unroll=8)
        return jnp.maximum(accp - accn + F32(MARGIN), F32(0.0))

    issue(0, 0)
    issue(1, 1)

    def pair(g2, loss):
        ga = g2 * 2
        for cp in copies(ga, 0):
            cp.wait()
        loss = loss + compute(ga, 0)

        @pl.when(ga + 2 < NCHUNK)
        def _():
            issue(ga + 2, 0)

        for cp in copies(ga + 1, 1):
            cp.wait()
        loss = loss + compute(ga + 1, 1)

        @pl.when(ga + 3 < NCHUNK)
        def _():
            issue(ga + 3, 1)
        return loss

    loss_acc = lax.fori_loop(0, NCHUNK // 2, pair, jnp.zeros((LANES,), F32))

    total = jnp.sum(loss_acc)
    out_stage[...] = jnp.where(lanes == 0, total, F32(0.0))
    pltpu.sync_copy(out_stage, out_hbm.at[pl.ds(wid * LANES, LANES)])


@jax.jit
def _launch(ent2, rel2, nv2, ph, pt, pr, nh, nt, nr):
    main = pl.kernel(
        _main_body,
        out_type=jax.ShapeDtypeStruct((NW * LANES,), F32),
        mesh=plsc.VectorSubcoreMesh(
            core_axis_name="c", subcore_axis_name="s",
            num_cores=NC, num_subcores=NS),
        compiler_params=pltpu.CompilerParams(needs_layout_passes=False,
                                             use_tc_tiling_on_sc=True),
        scratch_types=[pltpu.VMEM((PER_W,), I32)] * 12
        + [pltpu.VMEM((2, CHUNK, PADW), F32)] * 8
        + [pltpu.VMEM((LANES,), F32),
           pltpu.SemaphoreType.DMA, pltpu.SemaphoreType.DMA],
    )
    return main(ent2, rel2, nv2, ph, pt, pr, nh, nt, nr)


def kernel(pos_h, pos_t, pos_r, neg_h, neg_t, neg_r,
           ent_embeddings, rel_embeddings, normal_vectors):
    partials = _launch(
        ent_embeddings.reshape(ENT_TOTAL // 2, PADW),
        rel_embeddings.reshape(REL_TOTAL // 2, PADW),
        normal_vectors.reshape(REL_TOTAL // 2, PADW),
        pos_h.astype(I32), pos_t.astype(I32), pos_r.astype(I32),
        neg_h.astype(I32), neg_t.astype(I32), neg_r.astype(I32))
    return jnp.sum(partials)
